# Initial kernel scaffold; baseline (speedup 1.0000x reference)
import jax
import jax.numpy as jnp
from jax.experimental import pallas as pl


def kernel(x, rows, cols, values, bias):
    raise NotImplementedError("write your pallas kernel here")


# speedup vs baseline: 2.9412x; 2.9412x over previous
import functools
import math

import jax
import jax.numpy as jnp
from jax import lax
from jax.experimental import pallas as pl
from jax.experimental.pallas import tpu as pltpu
from jax.experimental.pallas import tpu_sc as plsc

N_IN = 16384
N_OUT = 16384
N_COLS = 256
W = 64
NCHUNK = N_COLS // W
NC = 2
NS = 16
L = 16
K = 128
RPT = N_OUT // NS
RB = 128


@functools.lru_cache(maxsize=None)
def _build(nnz_pad: int):
    nb = nnz_pad // (NS * K)
    tile_span = nb * K
    mesh = plsc.VectorSubcoreMesh(
        core_axis_name="c", subcore_axis_name="s",
        num_cores=NC, num_subcores=NS)

    @functools.partial(
        pl.kernel,
        out_type=jax.ShapeDtypeStruct((N_OUT, NCHUNK, W), jnp.float32),
        mesh=mesh,
        compiler_params=pltpu.CompilerParams(use_tc_tiling_on_sc=False),
        scratch_types=[
            pltpu.VMEM((K,), jnp.int32),
            pltpu.VMEM((K,), jnp.int32),
            pltpu.VMEM((K,), jnp.float32),
            pltpu.VMEM((K,), jnp.int32),
            pltpu.VMEM((K, W), jnp.float32),
            pltpu.VMEM((RB, W), jnp.float32),
            pltpu.VMEM((RPT,), jnp.float32),
            pltpu.VMEM_SHARED((N_OUT, W), jnp.float32),
            pltpu.SemaphoreType.DMA,
        ],
    )
    def sc_kernel(xv_hbm, rows_hbm, cols_hbm, vals_hbm, bias_hbm, out_hbm,
                  cols_v, rows_v, vals_v, idx_v, data_v, stage_v, bias_v,
                  acc, sem):
        cid = lax.axis_index("c")
        sid = lax.axis_index("s")
        r0 = sid * RPT
        base0 = sid * tile_span

        for qi in range(NCHUNK // NC):
            q = cid * (NCHUNK // NC) + qi

            def zrow(r, _):
                z = jnp.zeros((L,), jnp.float32)
                for c in range(W // L):
                    stage_v[r, pl.ds(c * L, L)] = z
                return 0
            lax.fori_loop(0, RB, zrow, 0)
            for blk in range(RPT // RB):
                pltpu.sync_copy(stage_v, acc.at[pl.ds(r0 + blk * RB, RB)])
            plsc.subcore_barrier()

            def batch(b, _):
                off = base0 + b * K
                pltpu.sync_copy(cols_hbm.at[pl.ds(off, K)], cols_v)
                pltpu.sync_copy(rows_hbm.at[pl.ds(off, K)], rows_v)
                pltpu.sync_copy(vals_hbm.at[pl.ds(off, K)], vals_v)
                for j in range(K // L):
                    sl = pl.ds(j * L, L)
                    idx_v[sl] = cols_v[sl] * NCHUNK + q
                pltpu.async_copy(xv_hbm.at[idx_v], data_v, sem).wait()

                def srow(g, _):
                    vvec = vals_v[pl.ds(g * L, L)]
                    for j in range(L):
                        k = g * L + j
                        v = vvec[j]
                        for c in range(W // L):
                            sl = pl.ds(c * L, L)
                            data_v[k, sl] = data_v[k, sl] * v
                    return 0
                lax.fori_loop(0, K // L, srow, 0)
                pltpu.sync_copy(data_v, acc.at[rows_v], add=True)
                return 0
            lax.fori_loop(0, nb, batch, 0)
            plsc.subcore_barrier()

            pltpu.sync_copy(bias_hbm.at[pl.ds(r0, RPT)], bias_v)
            for blk in range(RPT // RB):
                pltpu.sync_copy(acc.at[pl.ds(r0 + blk * RB, RB)], stage_v)

                def brow(g, _):
                    bvec = bias_v[pl.ds(blk * RB + g * L, L)]
                    for j in range(L):
                        r = g * L + j
                        bv = bvec[j]
                        for c in range(W // L):
                            sl = pl.ds(c * L, L)
                            stage_v[r, sl] = stage_v[r, sl] + bv
                    return 0
                lax.fori_loop(0, RB // L, brow, 0)
                pltpu.sync_copy(stage_v,
                                out_hbm.at[pl.ds(r0 + blk * RB, RB), q])

    return sc_kernel


def kernel(x, rows, cols, values, bias):
    x2 = x.reshape(x.shape[0], -1)
    nnz = values.shape[0]
    span = NS * K
    nnz_pad = math.ceil(nnz / span) * span
    pad = nnz_pad - nnz
    rows_p = jnp.pad(rows.astype(jnp.int32), (0, pad))
    cols_p = jnp.pad(cols.astype(jnp.int32), (0, pad))
    vals_p = jnp.pad(values, (0, pad))
    xv = x2.reshape(N_IN * NCHUNK, W)
    out = _build(nnz_pad)(xv, rows_p, cols_p, vals_p, bias.reshape(-1))
    shape = list(x.shape)
    shape[0] = N_OUT
    return out.reshape(shape)


# speedup vs baseline: 3.8182x; 1.2982x over previous
import functools
import math

import jax
import jax.numpy as jnp
from jax import lax
from jax.experimental import pallas as pl
from jax.experimental.pallas import tpu as pltpu
from jax.experimental.pallas import tpu_sc as plsc

N_IN = 16384
N_OUT = 16384
N_COLS = 256
W = 64
NCHUNK = N_COLS // W
NC = 2
NS = 16
L = 16
K = 128
RPT = N_OUT // NS
RB = 128


@functools.lru_cache(maxsize=None)
def _build(nnz_pad: int):
    nb = nnz_pad // (NS * K)
    nb2 = nb // 2
    mesh = plsc.VectorSubcoreMesh(
        core_axis_name="c", subcore_axis_name="s",
        num_cores=NC, num_subcores=NS)

    @functools.partial(
        pl.kernel,
        out_type=jax.ShapeDtypeStruct((N_OUT, NCHUNK, W), jnp.float32),
        mesh=mesh,
        compiler_params=pltpu.CompilerParams(use_tc_tiling_on_sc=False),
        scratch_types=[
            pltpu.VMEM((nb, K), jnp.int32),
            pltpu.VMEM((nb, K), jnp.float32),
            pltpu.VMEM((K,), jnp.int32),
            pltpu.VMEM((K,), jnp.int32),
            pltpu.VMEM((K,), jnp.int32),
            pltpu.VMEM((K,), jnp.int32),
            pltpu.VMEM((K, W), jnp.float32),
            pltpu.VMEM((K, W), jnp.float32),
            pltpu.VMEM((RB, W), jnp.float32),
            pltpu.VMEM((RPT,), jnp.float32),
            pltpu.VMEM_SHARED((N_OUT, W), jnp.float32),
            pltpu.SemaphoreType.DMA,
            pltpu.SemaphoreType.DMA,
            pltpu.SemaphoreType.DMA,
            pltpu.SemaphoreType.DMA,
        ],
    )
    def sc_kernel(xv_hbm, rows_hbm, cols_hbm, vals_hbm, bias_hbm, out_hbm,
                  cols_all, vals_all, rows0, rows1, idx0, idx1, buf0, buf1,
                  stage_v, bias_v, acc, semg0, semg1, semr0, semr1):
        cid = lax.axis_index("c")
        sid = lax.axis_index("s")
        r0 = sid * RPT

        pltpu.sync_copy(cols_hbm.at[pl.ds(sid * nb, nb)], cols_all)
        pltpu.sync_copy(vals_hbm.at[pl.ds(sid * nb, nb)], vals_all)
        pltpu.sync_copy(bias_hbm.at[pl.ds(r0, RPT)], bias_v)

        def compute_idx(bb, dst, q):
            for j in range(K // L):
                sl = pl.ds(j * L, L)
                dst[sl] = cols_all[bb, sl] * NCHUNK + q

        def scale(buf, bb):
            def srow(g, _):
                vvec = vals_all[bb, pl.ds(g * L, L)]
                for j in range(L):
                    k = g * L + j
                    v = vvec[j]
                    for c in range(W // L):
                        sl = pl.ds(c * L, L)
                        buf[k, sl] = buf[k, sl] * v
                return 0
            lax.fori_loop(0, K // L, srow, 0)

        def load_rows(bb, dst, sem):
            pltpu.async_copy(rows_hbm.at[sid * nb + bb], dst, sem)

        for qi in range(NCHUNK // NC):
            q = cid * (NCHUNK // NC) + qi

            def zrow(r, _):
                z = jnp.zeros((L,), jnp.float32)
                for c in range(W // L):
                    stage_v[r, pl.ds(c * L, L)] = z
                return 0
            lax.fori_loop(0, RB, zrow, 0)
            for blk in range(RPT // RB):
                pltpu.sync_copy(stage_v, acc.at[pl.ds(r0 + blk * RB, RB)])
            plsc.subcore_barrier()

            compute_idx(0, idx0, q)
            pltpu.async_copy(xv_hbm.at[idx0], buf0, semg0)
            load_rows(0, rows0, semr0)

            def body(b2, _):
                b = 2 * b2
                compute_idx(b + 1, idx1, q)
                pltpu.async_copy(xv_hbm.at[idx1], buf1, semg1)
                load_rows(b + 1, rows1, semr1)

                pltpu.make_async_copy(xv_hbm.at[idx0], buf0, semg0).wait()
                scale(buf0, b)
                pltpu.make_async_copy(rows_hbm.at[0], rows0, semr0).wait()
                pltpu.sync_copy(buf0, acc.at[rows0], add=True)

                @pl.when(b2 + 1 < nb2)
                def _():
                    compute_idx(b + 2, idx0, q)
                    pltpu.async_copy(xv_hbm.at[idx0], buf0, semg0)
                    load_rows(b + 2, rows0, semr0)

                pltpu.make_async_copy(xv_hbm.at[idx1], buf1, semg1).wait()
                scale(buf1, b + 1)
                pltpu.make_async_copy(rows_hbm.at[0], rows1, semr1).wait()
                pltpu.sync_copy(buf1, acc.at[rows1], add=True)
                return 0
            lax.fori_loop(0, nb2, body, 0)
            plsc.subcore_barrier()

            for blk in range(RPT // RB):
                pltpu.sync_copy(acc.at[pl.ds(r0 + blk * RB, RB)], stage_v)

                def brow(g, _):
                    bvec = bias_v[pl.ds(blk * RB + g * L, L)]
                    for j in range(L):
                        r = g * L + j
                        bv = bvec[j]
                        for c in range(W // L):
                            sl = pl.ds(c * L, L)
                            stage_v[r, sl] = stage_v[r, sl] + bv
                    return 0
                lax.fori_loop(0, RB // L, brow, 0)
                pltpu.sync_copy(stage_v,
                                out_hbm.at[pl.ds(r0 + blk * RB, RB), q])

    return sc_kernel


def kernel(x, rows, cols, values, bias):
    x2 = x.reshape(x.shape[0], -1)
    nnz = values.shape[0]
    span = NS * K
    nnz_pad = math.ceil(nnz / span) * span
    if (nnz_pad // span) % 2:
        nnz_pad += span
    pad = nnz_pad - nnz
    nbt = nnz_pad // (NS * K)
    rows_p = jnp.pad(rows.astype(jnp.int32), (0, pad)).reshape(NS * nbt, K)
    cols_p = jnp.pad(cols.astype(jnp.int32), (0, pad)).reshape(NS * nbt, K)
    vals_p = jnp.pad(values, (0, pad)).reshape(NS * nbt, K)
    xv = x2.reshape(N_IN * NCHUNK, W)
    out = _build(nnz_pad)(xv, rows_p, cols_p, vals_p, bias.reshape(-1))
    shape = list(x.shape)
    shape[0] = N_OUT
    return out.reshape(shape)


# speedup vs baseline: 7.0408x; 1.8440x over previous
import functools
import math

import jax
import jax.numpy as jnp
from jax import lax
from jax.experimental import pallas as pl
from jax.experimental.pallas import tpu as pltpu
from jax.experimental.pallas import tpu_sc as plsc

N_IN = 16384
N_OUT = 16384
N_COLS = 256
W = 64
NCHUNK = N_COLS // W
NC = 2
NS = 16
L = 16
K = 128
RPT = N_OUT // NS
RB = 128


@functools.lru_cache(maxsize=None)
def _build(nnz_pad: int):
    nb = nnz_pad // (NS * K)
    nb2 = nb // 2
    mesh = plsc.VectorSubcoreMesh(
        core_axis_name="c", subcore_axis_name="s",
        num_cores=NC, num_subcores=NS)

    @functools.partial(
        pl.kernel,
        out_type=jax.ShapeDtypeStruct((N_OUT, NCHUNK, W), jnp.float32),
        mesh=mesh,
        compiler_params=pltpu.CompilerParams(use_tc_tiling_on_sc=False),
        scratch_types=[
            pltpu.VMEM((nb, K), jnp.int32),
            pltpu.VMEM((nb, K), jnp.float32),
            pltpu.VMEM((K,), jnp.int32),
            pltpu.VMEM((K,), jnp.int32),
            pltpu.VMEM((K,), jnp.int32),
            pltpu.VMEM((K,), jnp.int32),
            pltpu.VMEM((K, W), jnp.float32),
            pltpu.VMEM((K, W), jnp.float32),
            pltpu.VMEM((RB, W), jnp.float32),
            pltpu.VMEM((RPT,), jnp.float32),
            pltpu.VMEM_SHARED((N_OUT, W), jnp.float32),
            pltpu.SemaphoreType.DMA,
            pltpu.SemaphoreType.DMA,
            pltpu.SemaphoreType.DMA,
            pltpu.SemaphoreType.DMA,
        ],
    )
    def sc_kernel(xv_hbm, rows_hbm, cols_hbm, vals_hbm, bias_hbm, out_hbm,
                  cols_all, vals_all, rows0, rows1, idx0, idx1, buf0, buf1,
                  stage_v, bias_v, acc, semg0, semg1, semr0, semr1):
        cid = lax.axis_index("c")
        sid = lax.axis_index("s")
        r0 = sid * RPT

        pltpu.sync_copy(cols_hbm.at[pl.ds(sid * nb, nb)], cols_all)
        pltpu.sync_copy(vals_hbm.at[pl.ds(sid * nb, nb)], vals_all)
        pltpu.sync_copy(bias_hbm.at[pl.ds(r0, RPT)], bias_v)

        def compute_idx(bb, dst, q):
            for j in range(K // L):
                sl = pl.ds(j * L, L)
                dst[sl] = cols_all[bb, sl] * NCHUNK + q

        def scale(buf, bb):
            def srow(g, _):
                vvec = vals_all[bb, pl.ds(g * L, L)]
                for j in range(L):
                    k = g * L + j
                    v = vvec[j]
                    for c in range(W // L):
                        sl = pl.ds(c * L, L)
                        buf[k, sl] = buf[k, sl] * v
                return 0
            lax.fori_loop(0, K // L, srow, 0, unroll=4)

        def load_rows(bb, dst, sem):
            pltpu.async_copy(rows_hbm.at[sid * nb + bb], dst, sem)

        for qi in range(NCHUNK // NC):
            q = cid * (NCHUNK // NC) + qi

            def zrow(r, _):
                z = jnp.zeros((L,), jnp.float32)
                for c in range(W // L):
                    stage_v[r, pl.ds(c * L, L)] = z
                return 0
            lax.fori_loop(0, RB, zrow, 0)
            for blk in range(RPT // RB):
                pltpu.sync_copy(stage_v, acc.at[pl.ds(r0 + blk * RB, RB)])
            plsc.subcore_barrier()

            compute_idx(0, idx0, q)
            pltpu.async_copy(xv_hbm.at[idx0], buf0, semg0)
            load_rows(0, rows0, semr0)

            def body(b2, _):
                b = 2 * b2
                compute_idx(b + 1, idx1, q)
                pltpu.async_copy(xv_hbm.at[idx1], buf1, semg1)
                load_rows(b + 1, rows1, semr1)

                pltpu.make_async_copy(xv_hbm.at[idx0], buf0, semg0).wait()
                scale(buf0, b)
                pltpu.make_async_copy(rows_hbm.at[0], rows0, semr0).wait()
                pltpu.sync_copy(buf0, acc.at[rows0], add=True)

                @pl.when(b2 + 1 < nb2)
                def _():
                    compute_idx(b + 2, idx0, q)
                    pltpu.async_copy(xv_hbm.at[idx0], buf0, semg0)
                    load_rows(b + 2, rows0, semr0)

                pltpu.make_async_copy(xv_hbm.at[idx1], buf1, semg1).wait()
                scale(buf1, b + 1)
                pltpu.make_async_copy(rows_hbm.at[0], rows1, semr1).wait()
                pltpu.sync_copy(buf1, acc.at[rows1], add=True)
                return 0
            lax.fori_loop(0, nb2, body, 0)
            plsc.subcore_barrier()

            for blk in range(RPT // RB):
                pltpu.sync_copy(acc.at[pl.ds(r0 + blk * RB, RB)], stage_v)

                def brow(g, _):
                    bvec = bias_v[pl.ds(blk * RB + g * L, L)]
                    for j in range(L):
                        r = g * L + j
                        bv = bvec[j]
                        for c in range(W // L):
                            sl = pl.ds(c * L, L)
                            stage_v[r, sl] = stage_v[r, sl] + bv
                    return 0
                lax.fori_loop(0, RB // L, brow, 0)
                pltpu.sync_copy(stage_v,
                                out_hbm.at[pl.ds(r0 + blk * RB, RB), q])

    return sc_kernel


def kernel(x, rows, cols, values, bias):
    x2 = x.reshape(x.shape[0], -1)
    nnz = values.shape[0]
    span = NS * K
    nnz_pad = math.ceil(nnz / span) * span
    if (nnz_pad // span) % 2:
        nnz_pad += span
    pad = nnz_pad - nnz
    nbt = nnz_pad // (NS * K)
    rows_p = jnp.pad(rows.astype(jnp.int32), (0, pad)).reshape(NS * nbt, K)
    cols_p = jnp.pad(cols.astype(jnp.int32), (0, pad)).reshape(NS * nbt, K)
    vals_p = jnp.pad(values, (0, pad)).reshape(NS * nbt, K)
    xv = x2.reshape(N_IN * NCHUNK, W)
    out = _build(nnz_pad)(xv, rows_p, cols_p, vals_p, bias.reshape(-1))
    shape = list(x.shape)
    shape[0] = N_OUT
    return out.reshape(shape)


# speedup vs baseline: 7.0679x; 1.0039x over previous
import functools
import math

import jax
import jax.numpy as jnp
from jax import lax
from jax.experimental import pallas as pl
from jax.experimental.pallas import tpu as pltpu
from jax.experimental.pallas import tpu_sc as plsc

N_IN = 16384
N_OUT = 16384
N_COLS = 256
W = 64
NCHUNK = N_COLS // W
NC = 2
NS = 16
L = 16
K = 128
RPT = N_OUT // NS
RB = 128


@functools.lru_cache(maxsize=None)
def _build(nnz_pad: int):
    nb = nnz_pad // (NS * K)
    nb2 = nb // 2
    mesh = plsc.VectorSubcoreMesh(
        core_axis_name="c", subcore_axis_name="s",
        num_cores=NC, num_subcores=NS)

    @functools.partial(
        pl.kernel,
        out_type=jax.ShapeDtypeStruct((N_OUT, NCHUNK, W), jnp.float32),
        mesh=mesh,
        compiler_params=pltpu.CompilerParams(use_tc_tiling_on_sc=False),
        scratch_types=[
            pltpu.VMEM((nb, K), jnp.int32),
            pltpu.VMEM((nb, K), jnp.float32),
            pltpu.VMEM((K,), jnp.int32),
            pltpu.VMEM((K,), jnp.int32),
            pltpu.VMEM((K,), jnp.int32),
            pltpu.VMEM((K,), jnp.int32),
            pltpu.VMEM((K, W), jnp.float32),
            pltpu.VMEM((K, W), jnp.float32),
            pltpu.VMEM((RB, W), jnp.float32),
            pltpu.VMEM((RPT,), jnp.float32),
            pltpu.VMEM_SHARED((N_OUT, W), jnp.float32),
            pltpu.SemaphoreType.DMA,
            pltpu.SemaphoreType.DMA,
            pltpu.SemaphoreType.DMA,
            pltpu.SemaphoreType.DMA,
        ],
    )
    def sc_kernel(xv_hbm, rows_hbm, cols_hbm, vals_hbm, bias_hbm, out_hbm,
                  cols_all, vals_all, rows0, rows1, idx0, idx1, buf0, buf1,
                  stage_v, bias_v, acc, semg0, semg1, semr0, semr1):
        cid = lax.axis_index("c")
        sid = lax.axis_index("s")
        r0 = sid * RPT

        pltpu.sync_copy(cols_hbm.at[pl.ds(sid * nb, nb)], cols_all)
        pltpu.sync_copy(vals_hbm.at[pl.ds(sid * nb, nb)], vals_all)
        pltpu.sync_copy(bias_hbm.at[pl.ds(r0, RPT)], bias_v)

        def compute_idx(bb, dst, q):
            for j in range(K // L):
                sl = pl.ds(j * L, L)
                dst[sl] = cols_all[bb, sl] * NCHUNK + q

        def scale(buf, bb):
            def srow(g, _):
                vvec = vals_all[bb, pl.ds(g * L, L)]
                for j in range(L):
                    k = g * L + j
                    v = vvec[j]
                    for c in range(W // L):
                        sl = pl.ds(c * L, L)
                        buf[k, sl] = buf[k, sl] * v
                return 0
            lax.fori_loop(0, K // L, srow, 0, unroll=4)

        def load_rows(bb, dst, sem):
            pltpu.async_copy(rows_hbm.at[sid * nb + bb], dst, sem)

        for qi in range(NCHUNK // NC):
            q = cid * (NCHUNK // NC) + qi

            def zrow(r, _):
                z = jnp.zeros((L,), jnp.float32)
                for c in range(W // L):
                    stage_v[r, pl.ds(c * L, L)] = z
                return 0
            lax.fori_loop(0, RB, zrow, 0, unroll=8)
            for blk in range(RPT // RB):
                pltpu.sync_copy(stage_v, acc.at[pl.ds(r0 + blk * RB, RB)])
            plsc.subcore_barrier()

            compute_idx(0, idx0, q)
            pltpu.async_copy(xv_hbm.at[idx0], buf0, semg0)
            load_rows(0, rows0, semr0)

            def body(b2, _):
                b = 2 * b2
                compute_idx(b + 1, idx1, q)
                pltpu.async_copy(xv_hbm.at[idx1], buf1, semg1)
                load_rows(b + 1, rows1, semr1)

                pltpu.make_async_copy(xv_hbm.at[idx0], buf0, semg0).wait()
                scale(buf0, b)
                pltpu.make_async_copy(rows_hbm.at[0], rows0, semr0).wait()
                pltpu.sync_copy(buf0, acc.at[rows0], add=True)

                @pl.when(b2 + 1 < nb2)
                def _():
                    compute_idx(b + 2, idx0, q)
                    pltpu.async_copy(xv_hbm.at[idx0], buf0, semg0)
                    load_rows(b + 2, rows0, semr0)

                pltpu.make_async_copy(xv_hbm.at[idx1], buf1, semg1).wait()
                scale(buf1, b + 1)
                pltpu.make_async_copy(rows_hbm.at[0], rows1, semr1).wait()
                pltpu.sync_copy(buf1, acc.at[rows1], add=True)
                return 0
            lax.fori_loop(0, nb2, body, 0)
            plsc.subcore_barrier()

            for blk in range(RPT // RB):
                pltpu.sync_copy(acc.at[pl.ds(r0 + blk * RB, RB)], stage_v)

                def brow(g, _):
                    bvec = bias_v[pl.ds(blk * RB + g * L, L)]
                    for j in range(L):
                        r = g * L + j
                        bv = bvec[j]
                        for c in range(W // L):
                            sl = pl.ds(c * L, L)
                            stage_v[r, sl] = stage_v[r, sl] + bv
                    return 0
                lax.fori_loop(0, RB // L, brow, 0, unroll=4)
                pltpu.sync_copy(stage_v,
                                out_hbm.at[pl.ds(r0 + blk * RB, RB), q])

    return sc_kernel


def kernel(x, rows, cols, values, bias):
    x2 = x.reshape(x.shape[0], -1)
    nnz = values.shape[0]
    span = NS * K
    nnz_pad = math.ceil(nnz / span) * span
    if (nnz_pad // span) % 2:
        nnz_pad += span
    pad = nnz_pad - nnz
    nbt = nnz_pad // (NS * K)
    rows_p = jnp.pad(rows.astype(jnp.int32), (0, pad)).reshape(NS * nbt, K)
    cols_p = jnp.pad(cols.astype(jnp.int32), (0, pad)).reshape(NS * nbt, K)
    vals_p = jnp.pad(values, (0, pad)).reshape(NS * nbt, K)
    xv = x2.reshape(N_IN * NCHUNK, W)
    out = _build(nnz_pad)(xv, rows_p, cols_p, vals_p, bias.reshape(-1))
    shape = list(x.shape)
    shape[0] = N_OUT
    return out.reshape(shape)


# speedup vs baseline: 7.4287x; 1.0510x over previous
import functools
import math

import jax
import jax.numpy as jnp
from jax import lax
from jax.experimental import pallas as pl
from jax.experimental.pallas import tpu as pltpu
from jax.experimental.pallas import tpu_sc as plsc

N_IN = 16384
N_OUT = 16384
N_COLS = 256
W = 64
NCHUNK = N_COLS // W
NC = 2
NS = 16
L = 16
K = 128
NBUF = 3
RPT = N_OUT // NS
RB = 128


@functools.lru_cache(maxsize=None)
def _build(span: int):
    nb = span // K
    mesh = plsc.VectorSubcoreMesh(
        core_axis_name="c", subcore_axis_name="s",
        num_cores=NC, num_subcores=NS)

    slot_types = []
    for _ in range(NBUF):
        slot_types += [
            pltpu.VMEM((K,), jnp.int32),
            pltpu.VMEM((K,), jnp.float32),
            pltpu.VMEM((K,), jnp.int32),
            pltpu.VMEM((K, W), jnp.float32),
            pltpu.SemaphoreType.DMA,
            pltpu.SemaphoreType.DMA,
            pltpu.SemaphoreType.DMA,
        ]

    @functools.partial(
        pl.kernel,
        out_type=jax.ShapeDtypeStruct((N_OUT, NCHUNK, W), jnp.float32),
        mesh=mesh,
        compiler_params=pltpu.CompilerParams(use_tc_tiling_on_sc=False),
        scratch_types=[
            pltpu.VMEM((span,), jnp.int32),
            pltpu.VMEM((RB, W), jnp.float32),
            pltpu.VMEM((RPT,), jnp.float32),
            pltpu.VMEM_SHARED((N_OUT, W), jnp.float32),
        ] + slot_types,
    )
    def sc_kernel(xv_hbm, rows_hbm, cols_hbm, vals_hbm,
                  rows_t_hbm, cols_t_hbm, vals_t_hbm, bias_hbm, out_hbm,
                  cols_all, stage_v, bias_v, acc, *slots):
        rows_s = [slots[7 * i + 0] for i in range(NBUF)]
        vals_s = [slots[7 * i + 1] for i in range(NBUF)]
        idx_s = [slots[7 * i + 2] for i in range(NBUF)]
        data_s = [slots[7 * i + 3] for i in range(NBUF)]
        semg = [slots[7 * i + 4] for i in range(NBUF)]
        sems = [slots[7 * i + 5] for i in range(NBUF)]
        semrv = [slots[7 * i + 6] for i in range(NBUF)]

        cid = lax.axis_index("c")
        sid = lax.axis_index("s")
        r0 = sid * RPT
        last = sid == NS - 1

        @pl.when(jnp.logical_not(last))
        def _():
            pltpu.sync_copy(cols_hbm.at[pl.ds(sid * span, span)], cols_all)

        @pl.when(last)
        def _():
            pltpu.sync_copy(cols_t_hbm, cols_all)

        pltpu.sync_copy(bias_hbm.at[pl.ds(r0, RPT)], bias_v)

        def compute_idx(bb, dst, q):
            for j in range(K // L):
                dst[pl.ds(j * L, L)] = (
                    cols_all[pl.ds(bb * K + j * L, L)] * NCHUNK + q)

        def load_rv(bb, s):
            @pl.when(jnp.logical_not(last))
            def _():
                off = sid * span + bb * K
                pltpu.async_copy(rows_hbm.at[pl.ds(off, K)], rows_s[s],
                                 semrv[s])
                pltpu.async_copy(vals_hbm.at[pl.ds(off, K)], vals_s[s],
                                 semrv[s])

            @pl.when(last)
            def _():
                pltpu.async_copy(rows_t_hbm.at[pl.ds(bb * K, K)], rows_s[s],
                                 semrv[s])
                pltpu.async_copy(vals_t_hbm.at[pl.ds(bb * K, K)], vals_s[s],
                                 semrv[s])

        def wait_rv(s):
            pltpu.make_async_copy(rows_hbm.at[pl.ds(0, K)], rows_s[s],
                                  semrv[s]).wait()
            pltpu.make_async_copy(vals_hbm.at[pl.ds(0, K)], vals_s[s],
                                  semrv[s]).wait()

        def scale(buf, vals_v):
            def srow(g, _):
                vvec = vals_v[pl.ds(g * L, L)]
                for j in range(L):
                    k = g * L + j
                    v = vvec[j]
                    for c in range(W // L):
                        sl = pl.ds(c * L, L)
                        buf[k, sl] = buf[k, sl] * v
                return 0
            lax.fori_loop(0, K // L, srow, 0, unroll=4)

        for qi in range(NCHUNK // NC):
            q = cid * (NCHUNK // NC) + qi

            def zrow(r, _):
                z = jnp.zeros((L,), jnp.float32)
                for c in range(W // L):
                    stage_v[r, pl.ds(c * L, L)] = z
                return 0
            lax.fori_loop(0, RB, zrow, 0, unroll=8)
            for blk in range(RPT // RB):
                pltpu.sync_copy(stage_v, acc.at[pl.ds(r0 + blk * RB, RB)])
            plsc.subcore_barrier()

            compute_idx(0, idx_s[0], q)
            pltpu.async_copy(xv_hbm.at[idx_s[0]], data_s[0], semg[0])
            load_rv(0, 0)

            def body3(n3, _):
                for s in range(NBUF):
                    n = n3 * NBUF + s
                    sp = (s + 1) % NBUF

                    @pl.when(n + 1 < nb)
                    def _():
                        compute_idx(n + 1, idx_s[sp], q)

                        @pl.when(n >= NBUF - 1)
                        def _():
                            pltpu.make_async_copy(
                                data_s[sp], acc.at[rows_s[sp]],
                                sems[sp]).wait()
                        pltpu.async_copy(xv_hbm.at[idx_s[sp]], data_s[sp],
                                         semg[sp])
                        load_rv(n + 1, sp)

                    pltpu.make_async_copy(xv_hbm.at[idx_s[s]], data_s[s],
                                          semg[s]).wait()
                    wait_rv(s)
                    scale(data_s[s], vals_s[s])
                    pltpu.async_copy(data_s[s], acc.at[rows_s[s]], sems[s],
                                     add=True)
                return 0
            lax.fori_loop(0, nb // NBUF, body3, 0)

            for s in range(NBUF):
                pltpu.make_async_copy(data_s[s], acc.at[rows_s[s]],
                                      sems[s]).wait()
            plsc.subcore_barrier()

            for blk in range(RPT // RB):
                pltpu.sync_copy(acc.at[pl.ds(r0 + blk * RB, RB)], stage_v)

                def brow(g, _):
                    bvec = bias_v[pl.ds(blk * RB + g * L, L)]
                    for j in range(L):
                        r = g * L + j
                        bv = bvec[j]
                        for c in range(W // L):
                            sl = pl.ds(c * L, L)
                            stage_v[r, sl] = stage_v[r, sl] + bv
                    return 0
                lax.fori_loop(0, RB // L, brow, 0, unroll=4)
                pltpu.sync_copy(stage_v,
                                out_hbm.at[pl.ds(r0 + blk * RB, RB), q])

    return sc_kernel


def kernel(x, rows, cols, values, bias):
    x2 = x.reshape(x.shape[0], -1)
    nnz = values.shape[0]
    span = math.ceil(nnz / (NS * K * NBUF)) * K * NBUF
    t0 = (NS - 1) * span
    rows_i = rows.astype(jnp.int32)
    cols_i = cols.astype(jnp.int32)
    tail = span - (nnz - t0)
    rows_t = jnp.pad(rows_i[t0:], (0, tail))
    cols_t = jnp.pad(cols_i[t0:], (0, tail))
    vals_t = jnp.pad(values[t0:], (0, tail))
    xv = x2.reshape(N_IN * NCHUNK, W)
    out = _build(span)(xv, rows_i, cols_i, values,
                       rows_t, cols_t, vals_t, bias.reshape(-1))
    shape = list(x.shape)
    shape[0] = N_OUT
    return out.reshape(shape)


# speedup vs baseline: 7.4910x; 1.0084x over previous
import functools
import math

import jax
import jax.numpy as jnp
from jax import lax
from jax.experimental import pallas as pl
from jax.experimental.pallas import tpu as pltpu
from jax.experimental.pallas import tpu_sc as plsc

N_IN = 16384
N_OUT = 16384
N_COLS = 256
W = 64
NCHUNK = N_COLS // W
NC = 2
NS = 16
L = 16
K = 128
NBUF = 3
RPT = N_OUT // NS
RB = 128


@functools.lru_cache(maxsize=None)
def _build(span: int):
    nb = span // K
    mesh = plsc.VectorSubcoreMesh(
        core_axis_name="c", subcore_axis_name="s",
        num_cores=NC, num_subcores=NS)

    slot_types = []
    for _ in range(NBUF):
        slot_types += [
            pltpu.VMEM((K,), jnp.int32),
            pltpu.VMEM((K,), jnp.float32),
            pltpu.VMEM((K,), jnp.int32),
            pltpu.VMEM((K, W), jnp.float32),
            pltpu.SemaphoreType.DMA,
            pltpu.SemaphoreType.DMA,
            pltpu.SemaphoreType.DMA,
        ]

    @functools.partial(
        pl.kernel,
        out_type=jax.ShapeDtypeStruct((N_OUT, NCHUNK, W), jnp.float32),
        mesh=mesh,
        compiler_params=pltpu.CompilerParams(use_tc_tiling_on_sc=False),
        scratch_types=[
            pltpu.VMEM((span,), jnp.int32),
            pltpu.VMEM((RB, W), jnp.float32),
            pltpu.VMEM((RPT,), jnp.float32),
            pltpu.VMEM_SHARED((N_OUT, W), jnp.float32),
        ] + slot_types,
    )
    def sc_kernel(xv_hbm, rows_hbm, cols_hbm, vals_hbm,
                  rows_t_hbm, cols_t_hbm, vals_t_hbm, bias_hbm, out_hbm,
                  cols_all, stage_v, bias_v, acc, *slots):
        rows_s = [slots[7 * i + 0] for i in range(NBUF)]
        vals_s = [slots[7 * i + 1] for i in range(NBUF)]
        idx_s = [slots[7 * i + 2] for i in range(NBUF)]
        data_s = [slots[7 * i + 3] for i in range(NBUF)]
        semg = [slots[7 * i + 4] for i in range(NBUF)]
        sems = [slots[7 * i + 5] for i in range(NBUF)]
        semrv = [slots[7 * i + 6] for i in range(NBUF)]

        cid = lax.axis_index("c")
        sid = lax.axis_index("s")
        r0 = sid * RPT
        last = sid == NS - 1

        @pl.when(jnp.logical_not(last))
        def _():
            pltpu.sync_copy(cols_hbm.at[pl.ds(sid * span, span)], cols_all)

        @pl.when(last)
        def _():
            pltpu.sync_copy(cols_t_hbm, cols_all)

        pltpu.sync_copy(bias_hbm.at[pl.ds(r0, RPT)], bias_v)

        def compute_idx(bb, dst, q):
            for j in range(K // L):
                dst[pl.ds(j * L, L)] = (
                    cols_all[pl.ds(bb * K + j * L, L)] * NCHUNK + q)

        def load_rv(bb, s):
            @pl.when(jnp.logical_not(last))
            def _():
                off = sid * span + bb * K
                pltpu.async_copy(rows_hbm.at[pl.ds(off, K)], rows_s[s],
                                 semrv[s])
                pltpu.async_copy(vals_hbm.at[pl.ds(off, K)], vals_s[s],
                                 semrv[s])

            @pl.when(last)
            def _():
                pltpu.async_copy(rows_t_hbm.at[pl.ds(bb * K, K)], rows_s[s],
                                 semrv[s])
                pltpu.async_copy(vals_t_hbm.at[pl.ds(bb * K, K)], vals_s[s],
                                 semrv[s])

        def wait_rv(s):
            pltpu.make_async_copy(rows_hbm.at[pl.ds(0, K)], rows_s[s],
                                  semrv[s]).wait()
            pltpu.make_async_copy(vals_hbm.at[pl.ds(0, K)], vals_s[s],
                                  semrv[s]).wait()

        def scale(buf, vals_v):
            def srow(g, _):
                vvec = vals_v[pl.ds(g * L, L)]
                for j in range(L):
                    k = g * L + j
                    v = vvec[j]
                    for c in range(W // L):
                        sl = pl.ds(c * L, L)
                        buf[k, sl] = buf[k, sl] * v
                return 0
            lax.fori_loop(0, K // L, srow, 0, unroll=4)

        for qi in range(NCHUNK // NC):
            q = cid * (NCHUNK // NC) + qi

            for blk in range(RPT // RB):
                def irow(g, _):
                    bvec = bias_v[pl.ds(blk * RB + g * L, L)]
                    for j in range(L):
                        r = g * L + j
                        brow = jnp.full((L,), bvec[j], jnp.float32)
                        for c in range(W // L):
                            stage_v[r, pl.ds(c * L, L)] = brow
                    return 0
                lax.fori_loop(0, RB // L, irow, 0, unroll=4)
                pltpu.sync_copy(stage_v, acc.at[pl.ds(r0 + blk * RB, RB)])
            plsc.subcore_barrier()

            compute_idx(0, idx_s[0], q)
            pltpu.async_copy(xv_hbm.at[idx_s[0]], data_s[0], semg[0])
            load_rv(0, 0)

            def body3(n3, _):
                for s in range(NBUF):
                    n = n3 * NBUF + s
                    sp = (s + 1) % NBUF

                    @pl.when(n + 1 < nb)
                    def _():
                        compute_idx(n + 1, idx_s[sp], q)

                        @pl.when(n >= NBUF - 1)
                        def _():
                            pltpu.make_async_copy(
                                data_s[sp], acc.at[rows_s[sp]],
                                sems[sp]).wait()
                        pltpu.async_copy(xv_hbm.at[idx_s[sp]], data_s[sp],
                                         semg[sp])
                        load_rv(n + 1, sp)

                    pltpu.make_async_copy(xv_hbm.at[idx_s[s]], data_s[s],
                                          semg[s]).wait()
                    wait_rv(s)
                    scale(data_s[s], vals_s[s])
                    pltpu.async_copy(data_s[s], acc.at[rows_s[s]], sems[s],
                                     add=True)
                return 0
            lax.fori_loop(0, nb // NBUF, body3, 0)

            for s in range(NBUF):
                pltpu.make_async_copy(data_s[s], acc.at[rows_s[s]],
                                      sems[s]).wait()
            plsc.subcore_barrier()

            pltpu.sync_copy(acc.at[pl.ds(r0, RPT)],
                            out_hbm.at[pl.ds(r0, RPT), q])

    return sc_kernel


def kernel(x, rows, cols, values, bias):
    x2 = x.reshape(x.shape[0], -1)
    nnz = values.shape[0]
    span = math.ceil(nnz / (NS * K * NBUF)) * K * NBUF
    t0 = (NS - 1) * span
    rows_i = rows.astype(jnp.int32)
    cols_i = cols.astype(jnp.int32)
    tail = span - (nnz - t0)
    rows_t = jnp.pad(rows_i[t0:], (0, tail))
    cols_t = jnp.pad(cols_i[t0:], (0, tail))
    vals_t = jnp.pad(values[t0:], (0, tail))
    xv = x2.reshape(N_IN * NCHUNK, W)
    out = _build(span)(xv, rows_i, cols_i, values,
                       rows_t, cols_t, vals_t, bias.reshape(-1))
    shape = list(x.shape)
    shape[0] = N_OUT
    return out.reshape(shape)


# speedup vs baseline: 7.9265x; 1.0581x over previous
import functools
import math

import jax
import jax.numpy as jnp
from jax import lax
from jax.experimental import pallas as pl
from jax.experimental.pallas import tpu as pltpu
from jax.experimental.pallas import tpu_sc as plsc

N_IN = 16384
N_OUT = 16384
N_COLS = 256
W = 64
NCHUNK = N_COLS // W
NC = 2
NS = 16
L = 16
K = 128
NBUF = 4
RPT = N_OUT // NS
RB = 128


@functools.lru_cache(maxsize=None)
def _build(span: int):
    nb = span // K
    mesh = plsc.VectorSubcoreMesh(
        core_axis_name="c", subcore_axis_name="s",
        num_cores=NC, num_subcores=NS)

    slot_types = []
    for _ in range(NBUF):
        slot_types += [
            pltpu.VMEM((K,), jnp.int32),
            pltpu.VMEM((K,), jnp.float32),
            pltpu.VMEM((K,), jnp.int32),
            pltpu.VMEM((K, W), jnp.float32),
            pltpu.SemaphoreType.DMA,
            pltpu.SemaphoreType.DMA,
            pltpu.SemaphoreType.DMA,
        ]

    @functools.partial(
        pl.kernel,
        out_type=jax.ShapeDtypeStruct((N_OUT, NCHUNK, W), jnp.float32),
        mesh=mesh,
        compiler_params=pltpu.CompilerParams(use_tc_tiling_on_sc=False),
        scratch_types=[
            pltpu.VMEM((span,), jnp.int32),
            pltpu.VMEM((RB, W), jnp.float32),
            pltpu.VMEM((RPT,), jnp.float32),
            pltpu.VMEM_SHARED((N_OUT, W), jnp.float32),
        ] + slot_types,
    )
    def sc_kernel(xv_hbm, rows_hbm, cols_hbm, vals_hbm,
                  rows_t_hbm, cols_t_hbm, vals_t_hbm, bias_hbm, out_hbm,
                  cols_all, stage_v, bias_v, acc, *slots):
        rows_s = [slots[7 * i + 0] for i in range(NBUF)]
        vals_s = [slots[7 * i + 1] for i in range(NBUF)]
        idx_s = [slots[7 * i + 2] for i in range(NBUF)]
        data_s = [slots[7 * i + 3] for i in range(NBUF)]
        semg = [slots[7 * i + 4] for i in range(NBUF)]
        sems = [slots[7 * i + 5] for i in range(NBUF)]
        semrv = [slots[7 * i + 6] for i in range(NBUF)]

        cid = lax.axis_index("c")
        sid = lax.axis_index("s")
        r0 = sid * RPT
        last = sid == NS - 1

        @pl.when(jnp.logical_not(last))
        def _():
            pltpu.sync_copy(cols_hbm.at[pl.ds(sid * span, span)], cols_all)

        @pl.when(last)
        def _():
            pltpu.sync_copy(cols_t_hbm, cols_all)

        pltpu.sync_copy(bias_hbm.at[pl.ds(r0, RPT)], bias_v)

        def compute_idx(bb, dst, q):
            for j in range(K // L):
                dst[pl.ds(j * L, L)] = (
                    cols_all[pl.ds(bb * K + j * L, L)] * NCHUNK + q)

        def load_rv(bb, s):
            @pl.when(jnp.logical_not(last))
            def _():
                off = sid * span + bb * K
                pltpu.async_copy(rows_hbm.at[pl.ds(off, K)], rows_s[s],
                                 semrv[s])
                pltpu.async_copy(vals_hbm.at[pl.ds(off, K)], vals_s[s],
                                 semrv[s])

            @pl.when(last)
            def _():
                pltpu.async_copy(rows_t_hbm.at[pl.ds(bb * K, K)], rows_s[s],
                                 semrv[s])
                pltpu.async_copy(vals_t_hbm.at[pl.ds(bb * K, K)], vals_s[s],
                                 semrv[s])

        def wait_rv(s):
            pltpu.make_async_copy(rows_hbm.at[pl.ds(0, K)], rows_s[s],
                                  semrv[s]).wait()
            pltpu.make_async_copy(vals_hbm.at[pl.ds(0, K)], vals_s[s],
                                  semrv[s]).wait()

        def scale(buf, vals_v):
            def srow(g, _):
                vvec = vals_v[pl.ds(g * L, L)]
                for j in range(L):
                    k = g * L + j
                    v = vvec[j]
                    for c in range(W // L):
                        sl = pl.ds(c * L, L)
                        buf[k, sl] = buf[k, sl] * v
                return 0
            lax.fori_loop(0, K // L, srow, 0, unroll=4)

        for qi in range(NCHUNK // NC):
            q = cid * (NCHUNK // NC) + qi

            for blk in range(RPT // RB):
                def irow(g, _):
                    bvec = bias_v[pl.ds(blk * RB + g * L, L)]
                    for j in range(L):
                        r = g * L + j
                        brow = jnp.full((L,), bvec[j], jnp.float32)
                        for c in range(W // L):
                            stage_v[r, pl.ds(c * L, L)] = brow
                    return 0
                lax.fori_loop(0, RB // L, irow, 0, unroll=4)
                pltpu.sync_copy(stage_v, acc.at[pl.ds(r0 + blk * RB, RB)])
            plsc.subcore_barrier()

            for p in range(2):
                compute_idx(p, idx_s[p], q)
                pltpu.async_copy(xv_hbm.at[idx_s[p]], data_s[p], semg[p])
                load_rv(p, p)

            def body4(n4, _):
                for s in range(NBUF):
                    n = n4 * NBUF + s
                    s2 = (s + 2) % NBUF

                    @pl.when(n + 2 < nb)
                    def _():
                        compute_idx(n + 2, idx_s[s2], q)

                        @pl.when(n >= 2)
                        def _():
                            pltpu.make_async_copy(
                                data_s[s2], acc.at[rows_s[s2]],
                                sems[s2]).wait()
                        pltpu.async_copy(xv_hbm.at[idx_s[s2]], data_s[s2],
                                         semg[s2])
                        load_rv(n + 2, s2)

                    pltpu.make_async_copy(xv_hbm.at[idx_s[s]], data_s[s],
                                          semg[s]).wait()
                    wait_rv(s)
                    scale(data_s[s], vals_s[s])
                    pltpu.async_copy(data_s[s], acc.at[rows_s[s]], sems[s],
                                     add=True)
                return 0
            lax.fori_loop(0, nb // NBUF, body4, 0)

            for s in range(NBUF):
                pltpu.make_async_copy(data_s[s], acc.at[rows_s[s]],
                                      sems[s]).wait()
            plsc.subcore_barrier()

            pltpu.sync_copy(acc.at[pl.ds(r0, RPT)],
                            out_hbm.at[pl.ds(r0, RPT), q])

    return sc_kernel


def kernel(x, rows, cols, values, bias):
    x2 = x.reshape(x.shape[0], -1)
    nnz = values.shape[0]
    span = math.ceil(nnz / (NS * K * NBUF)) * K * NBUF
    t0 = (NS - 1) * span
    rows_i = rows.astype(jnp.int32)
    cols_i = cols.astype(jnp.int32)
    tail = span - (nnz - t0)
    rows_t = jnp.pad(rows_i[t0:], (0, tail))
    cols_t = jnp.pad(cols_i[t0:], (0, tail))
    vals_t = jnp.pad(values[t0:], (0, tail))
    xv = x2.reshape(N_IN * NCHUNK, W)
    out = _build(span)(xv, rows_i, cols_i, values,
                       rows_t, cols_t, vals_t, bias.reshape(-1))
    shape = list(x.shape)
    shape[0] = N_OUT
    return out.reshape(shape)
